# baseline (speedup 1.0000x reference)
import functools

import jax
import jax.numpy as jnp
from jax import lax
from jax.experimental import pallas as pl
from jax.experimental.pallas import tpu as pltpu
from jax.experimental.pallas import tpu_sc as plsc

N = 50000
E = 800000
G = 128
DIN_X = 29
DH_X = 64
DIN_E = 17
DH_E = 32
ER = E // 128
NB = 10000
NG = N // NB
EB = 6400
EG = E // EB
NPT = N // 16
F32 = jnp.float32


def _elu(v):
    return jnp.where(v > 0, v, jnp.exp(jnp.minimum(v, 0.0)) - 1.0)


def _embed_x_body(x_ref, w_ref, b_ref, o_ref, ob_ref):
    h = _elu(jnp.dot(x_ref[...], w_ref[...],
                     preferred_element_type=F32) + b_ref[...])
    o_ref[...] = h
    ob_ref[...] = h.astype(jnp.bfloat16)


def _embed_e_body(ea_ref, w_ref, b_ref, e_ref, cov_ref, non_ref):
    ea = ea_ref[...]
    e_ref[...] = _elu(jnp.dot(ea, w_ref[...],
                              preferred_element_type=F32) + b_ref[...])
    c = jnp.where(ea[:, 0:1] > 0.5, 1.0, 0.0)
    cov_ref[...] = c
    non_ref[...] = 1.0 - c


def _edge_msg_body(hd_ref, hs_ref, e_ref, m_ref, wfd_ref, wfs_ref, wfe_ref,
                   fb_ref, wsd_ref, wss_ref, wse_ref, sb_ref,
                   mlo_ref, mhi_ref):
    hd = hd_ref[...].astype(F32)
    hs = hs_ref[...].astype(F32)
    e = e_ref[...]
    f = (jnp.dot(hd, wfd_ref[...], preferred_element_type=F32)
         + jnp.dot(hs, wfs_ref[...], preferred_element_type=F32)
         + jnp.dot(e, wfe_ref[...], preferred_element_type=F32) + fb_ref[...])
    s = (jnp.dot(hd, wsd_ref[...], preferred_element_type=F32)
         + jnp.dot(hs, wss_ref[...], preferred_element_type=F32)
         + jnp.dot(e, wse_ref[...], preferred_element_type=F32) + sb_ref[...])
    sig = 1.0 / (1.0 + jnp.exp(-f))
    sp = jnp.maximum(s, 0.0) + jnp.log(1.0 + jnp.exp(-jnp.abs(s)))
    msg = m_ref[...] * sig * sp
    mlo_ref[...] = msg[:, :32]
    mhi_ref[...] = msg[:, 32:]


def _node_upd_body(alo_ref, ahi_ref, h_ref, lg_ref, lb_ref, lw_ref, lbias_ref,
                   o_ref, ob_ref):
    h = h_ref[...]
    v = jnp.concatenate([alo_ref[...], ahi_ref[...]], axis=1) + h
    m = jnp.mean(v, axis=1, keepdims=True)
    var = jnp.mean((v - m) * (v - m), axis=1, keepdims=True)
    vn = (v - m) / jnp.sqrt(var + 1e-5) * lg_ref[...] + lb_ref[...]
    o = jnp.dot(vn, lw_ref[...], preferred_element_type=F32) + lbias_ref[...] + h
    o = _elu(o)
    o_ref[...] = o
    ob_ref[...] = o.astype(jnp.bfloat16)


def _head_body(h_ref, x_ref, b_ref, w1_ref, b1_ref, w2_ref, b2_ref, tw_ref,
               na_ref, o_ref):
    pid = pl.program_id(0)
    t = _elu(jnp.dot(h_ref[...], w1_ref[...], preferred_element_type=F32)
             + b1_ref[...])
    t = _elu(jnp.dot(t, w2_ref[...], preferred_element_type=F32) + b2_ref[...])
    lig = jnp.where(x_ref[:, 0:1] > 0.5, 1.0, 0.0)
    sc = jnp.dot(t * lig, tw_ref[...], preferred_element_type=F32)
    onehot = (b_ref[...] == lax.broadcasted_iota(jnp.int32, (NB, G), 1)
              ).astype(F32)
    contrib = lax.dot_general(onehot, sc, (((0,), (0,)), ((), ())))

    @pl.when(pid == 0)
    def _():
        o_ref[...] = jnp.zeros_like(o_ref)

    o_ref[...] += contrib

    @pl.when(pid == NG - 1)
    def _():
        o_ref[...] = o_ref[...] / na_ref[...]


def _const2d(shape):
    return pl.BlockSpec(shape, lambda i: (0, 0))


def _embed_x(x, w_t, b):
    return pl.pallas_call(
        _embed_x_body,
        grid=(NG,),
        in_specs=[pl.BlockSpec((NB, DIN_X), lambda i: (i, 0)),
                  _const2d((DIN_X, DH_X)), _const2d((1, DH_X))],
        out_specs=[pl.BlockSpec((NB, DH_X), lambda i: (i, 0)),
                   pl.BlockSpec((NB, DH_X), lambda i: (i, 0))],
        out_shape=[jax.ShapeDtypeStruct((N, DH_X), F32),
                   jax.ShapeDtypeStruct((N, DH_X), jnp.bfloat16)],
    )(x, w_t, b)


def _embed_e(ea, w_t, b):
    return pl.pallas_call(
        _embed_e_body,
        grid=(EG,),
        in_specs=[pl.BlockSpec((EB, DIN_E), lambda i: (i, 0)),
                  _const2d((DIN_E, DH_E)), _const2d((1, DH_E))],
        out_specs=[pl.BlockSpec((EB, DH_E), lambda i: (i, 0)),
                   pl.BlockSpec((EB, 1), lambda i: (i, 0)),
                   pl.BlockSpec((EB, 1), lambda i: (i, 0))],
        out_shape=[jax.ShapeDtypeStruct((E, DH_E), F32),
                   jax.ShapeDtypeStruct((E, 1), F32),
                   jax.ShapeDtypeStruct((E, 1), F32)],
    )(ea, w_t, b)


def _edge_msg(hd, hs, e, m, wfd, wfs, wfe, fb, wsd, wss, wse, sb):
    return pl.pallas_call(
        _edge_msg_body,
        grid=(EG,),
        in_specs=[pl.BlockSpec((EB, DH_X), lambda i: (i, 0)),
                  pl.BlockSpec((EB, DH_X), lambda i: (i, 0)),
                  pl.BlockSpec((EB, DH_E), lambda i: (i, 0)),
                  pl.BlockSpec((EB, 1), lambda i: (i, 0)),
                  _const2d((DH_X, DH_X)), _const2d((DH_X, DH_X)),
                  _const2d((DH_E, DH_X)), _const2d((1, DH_X)),
                  _const2d((DH_X, DH_X)), _const2d((DH_X, DH_X)),
                  _const2d((DH_E, DH_X)), _const2d((1, DH_X))],
        out_specs=[pl.BlockSpec((EB, 32), lambda i: (i, 0)),
                   pl.BlockSpec((EB, 32), lambda i: (i, 0))],
        out_shape=[jax.ShapeDtypeStruct((E, 32), F32),
                   jax.ShapeDtypeStruct((E, 32), F32)],
    )(hd, hs, e, m, wfd, wfs, wfe, fb, wsd, wss, wse, sb)


def _node_upd(alo, ahi, h, lg, lb, lw_t, lbias):
    return pl.pallas_call(
        _node_upd_body,
        grid=(NG,),
        in_specs=[pl.BlockSpec((NB, 32), lambda i: (i, 0)),
                  pl.BlockSpec((NB, 32), lambda i: (i, 0)),
                  pl.BlockSpec((NB, DH_X), lambda i: (i, 0)),
                  _const2d((1, DH_X)), _const2d((1, DH_X)),
                  _const2d((DH_X, DH_X)), _const2d((1, DH_X))],
        out_specs=[pl.BlockSpec((NB, DH_X), lambda i: (i, 0)),
                   pl.BlockSpec((NB, DH_X), lambda i: (i, 0))],
        out_shape=[jax.ShapeDtypeStruct((N, DH_X), F32),
                   jax.ShapeDtypeStruct((N, DH_X), jnp.bfloat16)],
    )(alo, ahi, h, lg, lb, lw_t, lbias)


def _head(h, x, batch2d, w1_t, b1, w2_t, b2, tw_t, na):
    return pl.pallas_call(
        _head_body,
        grid=(NG,),
        in_specs=[pl.BlockSpec((NB, DH_X), lambda i: (i, 0)),
                  pl.BlockSpec((NB, DIN_X), lambda i: (i, 0)),
                  pl.BlockSpec((NB, 1), lambda i: (i, 0)),
                  _const2d((DH_X, 32)), _const2d((1, 32)),
                  _const2d((32, 16)), _const2d((1, 16)),
                  _const2d((16, 1)), _const2d((G, 1))],
        out_specs=pl.BlockSpec((G, 1), lambda i: (0, 0)),
        out_shape=jax.ShapeDtypeStruct((G, 1), F32),
    )(h, x, batch2d, w1_t, b1, w2_t, b2, tw_t, na)


_MESH = dict(core_axis_name="c", subcore_axis_name="s",
             num_cores=2, num_subcores=16)


def _sc_gather(h, dst2d, src2d):
    BF16 = jnp.bfloat16

    @functools.partial(
        pl.kernel,
        out_type=[jax.ShapeDtypeStruct((E, DH_X), BF16),
                  jax.ShapeDtypeStruct((E, DH_X), BF16)],
        mesh=plsc.VectorSubcoreMesh(**_MESH),
        compiler_params=pltpu.CompilerParams(use_tc_tiling_on_sc=False),
        scratch_types=[pltpu.VMEM((128,), jnp.int32),
                       pltpu.VMEM((128,), jnp.int32),
                       pltpu.VMEM((128,), jnp.int32),
                       pltpu.VMEM((128,), jnp.int32),
                       pltpu.VMEM((128, DH_X), BF16),
                       pltpu.VMEM((128, DH_X), BF16),
                       pltpu.VMEM((128, DH_X), BF16),
                       pltpu.VMEM((128, DH_X), BF16),
                       pltpu.SemaphoreType.DMA,
                       pltpu.SemaphoreType.DMA,
                       pltpu.SemaphoreType.DMA],
    )
    def gk(h_hbm, d2_hbm, s2_hbm, hd_out, hs_out, idx_d0, idx_s0, idx_d1,
           idx_s1, bufd0, bufs0, bufd1, bufs1, semi, semg, semw):
        w = lax.axis_index("s") * 2 + lax.axis_index("c")
        start = 195 * w + jnp.minimum(w, 10)
        cnt = jnp.where(w < 10, 196, 195)

        def row(j):
            return start + jnp.minimum(j, cnt - 1)

        pltpu.async_copy(d2_hbm.at[row(0)], idx_d0, semi)
        pltpu.async_copy(s2_hbm.at[row(0)], idx_s0, semi)
        pltpu.async_copy(d2_hbm.at[row(1)], idx_d1, semi)
        pltpu.async_copy(s2_hbm.at[row(1)], idx_s1, semi)

        def phase(j, idx_d, idx_s, bufd, bufs):
            r = row(j)
            o = pl.ds(r * 128, 128)
            pltpu.make_async_copy(d2_hbm.at[r], idx_d, semi).wait()
            pltpu.make_async_copy(s2_hbm.at[r], idx_s, semi).wait()

            @pl.when(j >= 2)
            def _():
                pltpu.make_async_copy(bufd, hd_out.at[o], semw).wait()
                pltpu.make_async_copy(bufs, hs_out.at[o], semw).wait()

            gd = pltpu.async_copy(h_hbm.at[idx_d], bufd, semg)
            gs = pltpu.async_copy(h_hbm.at[idx_s], bufs, semg)
            gd.wait()
            gs.wait()

            @pl.when(j + 2 < 196)
            def _():
                pltpu.async_copy(d2_hbm.at[row(j + 2)], idx_d, semi)
                pltpu.async_copy(s2_hbm.at[row(j + 2)], idx_s, semi)

            pltpu.async_copy(bufd, hd_out.at[o], semw)
            pltpu.async_copy(bufs, hs_out.at[o], semw)

        def body(jj, carry):
            phase(2 * jj, idx_d0, idx_s0, bufd0, bufs0)
            phase(2 * jj + 1, idx_d1, idx_s1, bufd1, bufs1)
            return carry

        lax.fori_loop(0, 98, body, 0)
        o0 = pl.ds(start * 128, 128)
        pltpu.make_async_copy(bufd0, hd_out.at[o0], semw).wait()
        pltpu.make_async_copy(bufs0, hs_out.at[o0], semw).wait()
        pltpu.make_async_copy(bufd1, hd_out.at[o0], semw).wait()
        pltpu.make_async_copy(bufs1, hs_out.at[o0], semw).wait()

    return gk(h, dst2d, src2d)


def _sc_scatter(mlo, mhi, dst2d, ztile):
    @functools.partial(
        pl.kernel,
        out_type=[jax.ShapeDtypeStruct((N, 32), F32),
                  jax.ShapeDtypeStruct((N, 32), F32)],
        mesh=plsc.VectorSubcoreMesh(**_MESH),
        compiler_params=pltpu.CompilerParams(use_tc_tiling_on_sc=False),
        scratch_types=[pltpu.VMEM((256, 32), F32),
                       pltpu.VMEM((256, 32), F32),
                       pltpu.VMEM((2, 128), jnp.int32),
                       pltpu.VMEM((2, 128), jnp.int32),
                       pltpu.VMEM_SHARED((N, 32), F32),
                       pltpu.SemaphoreType.DMA],
    )
    def sk(mlo_hbm, mhi_hbm, d2_hbm, z_hbm, alo_out, ahi_out,
           mbuf0, mbuf1, midx0, midx1, shared, seml):
        cc = lax.axis_index("c")
        t = lax.axis_index("s")
        pltpu.sync_copy(z_hbm, shared.at[pl.ds(t * NPT, NPT)])
        plsc.subcore_barrier()
        start = 390 * t + jnp.minimum(t, 10)
        cnt = jnp.where(t < 10, 391, 390)
        nblk = 196

        def lbase(b):
            return jnp.minimum(start + 2 * b, ER - 2)

        def issue(b, midx, mbuf):
            lb = lbase(b)
            pltpu.async_copy(d2_hbm.at[pl.ds(lb, 2)], midx, seml)

            @pl.when(cc == 0)
            def _():
                pltpu.async_copy(mlo_hbm.at[pl.ds(lb * 128, 256)], mbuf,
                                 seml)

            @pl.when(cc == 1)
            def _():
                pltpu.async_copy(mhi_hbm.at[pl.ds(lb * 128, 256)], mbuf,
                                 seml)

        issue(0, midx0, mbuf0)
        issue(1, midx1, mbuf1)

        def phase(b, midx, mbuf):
            lb = lbase(b)
            pltpu.make_async_copy(d2_hbm.at[pl.ds(lb, 2)], midx, seml).wait()
            pltpu.make_async_copy(mlo_hbm.at[pl.ds(lb * 128, 256)], mbuf,
                                  seml).wait()
            for k in range(2):
                r = lb + k

                @pl.when((r >= start + 2 * b) & (r < start + cnt))
                def _():
                    pltpu.sync_copy(mbuf.at[pl.ds(k * 128, 128)],
                                    shared.at[midx.at[k]], add=True)

            @pl.when(b + 2 < nblk)
            def _():
                issue(b + 2, midx, mbuf)

        def body(bb, carry):
            phase(2 * bb, midx0, mbuf0)
            phase(2 * bb + 1, midx1, mbuf1)
            return carry

        lax.fori_loop(0, nblk // 2, body, 0)
        plsc.subcore_barrier()

        @pl.when(cc == 0)
        def _():
            pltpu.sync_copy(shared.at[pl.ds(t * NPT, NPT)],
                            alo_out.at[pl.ds(t * NPT, NPT)])

        @pl.when(cc == 1)
        def _():
            pltpu.sync_copy(shared.at[pl.ds(t * NPT, NPT)],
                            ahi_out.at[pl.ds(t * NPT, NPT)])

    return sk(mlo, mhi, dst2d, ztile)


def kernel(x, edge_index, edge_attr, batch, n_atom, embed_x_w, embed_x_b,
           embed_e_w, embed_e_b, cov_lin_f_w, cov_lin_f_b, cov_lin_s_w,
           cov_lin_s_b, cov_ln_g, cov_ln_b, cov_lin_w, cov_lin_b, non_lin_f_w,
           non_lin_f_b, non_lin_s_w, non_lin_s_b, non_ln_g, non_ln_b,
           non_lin_w, non_lin_b, mlp_w1, mlp_b1, mlp_w2, mlp_b2, trans_w):
    dst2d = edge_index[1].reshape(ER, 128)
    src2d = edge_index[0].reshape(ER, 128)
    ztile = jnp.zeros((NPT, 32), F32)

    h, hb = _embed_x(x, embed_x_w.T, embed_x_b.reshape(1, -1))
    e, cov, non = _embed_e(edge_attr, embed_e_w.T, embed_e_b.reshape(1, -1))

    for (m, fw, fb, sw, sb, lg, lb, lw, lbias) in (
        (cov, cov_lin_f_w, cov_lin_f_b, cov_lin_s_w, cov_lin_s_b, cov_ln_g,
         cov_ln_b, cov_lin_w, cov_lin_b),
        (non, non_lin_f_w, non_lin_f_b, non_lin_s_w, non_lin_s_b, non_ln_g,
         non_ln_b, non_lin_w, non_lin_b),
    ):
        hd, hs = _sc_gather(hb, dst2d, src2d)
        mlo, mhi = _edge_msg(
            hd, hs, e, m,
            fw[:, :DH_X].T, fw[:, DH_X:2 * DH_X].T, fw[:, 2 * DH_X:].T,
            fb.reshape(1, -1),
            sw[:, :DH_X].T, sw[:, DH_X:2 * DH_X].T, sw[:, 2 * DH_X:].T,
            sb.reshape(1, -1))
        alo, ahi = _sc_scatter(mlo, mhi, dst2d, ztile)
        h, hb = _node_upd(alo, ahi, h, lg.reshape(1, -1), lb.reshape(1, -1),
                          lw.T, lbias.reshape(1, -1))

    energy = _head(h, x, batch.reshape(N, 1), mlp_w1.T, mlp_b1.reshape(1, -1),
                   mlp_w2.T, mlp_b2.reshape(1, -1), trans_w.T,
                   n_atom.reshape(G, 1))
    return energy.reshape(-1)


# speedup vs baseline: 1.0006x; 1.0006x over previous
import functools

import jax
import jax.numpy as jnp
from jax import lax
from jax.experimental import pallas as pl
from jax.experimental.pallas import tpu as pltpu
from jax.experimental.pallas import tpu_sc as plsc

N = 50000
E = 800000
G = 128
DIN_X = 29
DH_X = 64
DIN_E = 17
DH_E = 32
ER = E // 128
NB = 10000
NG = N // NB
EB = 6400
EG = E // EB
NPT = N // 16
F32 = jnp.float32


def _elu(v):
    return jnp.where(v > 0, v, jnp.exp(jnp.minimum(v, 0.0)) - 1.0)


def _embed_x_body(x_ref, w_ref, b_ref, o_ref, ob_ref):
    h = _elu(jnp.dot(x_ref[...], w_ref[...],
                     preferred_element_type=F32) + b_ref[...])
    o_ref[...] = h
    ob_ref[...] = h.astype(jnp.bfloat16)


def _embed_e_body(ea_ref, w_ref, b_ref, e_ref, cov_ref, non_ref):
    ea = ea_ref[...]
    e_ref[...] = _elu(jnp.dot(ea, w_ref[...],
                              preferred_element_type=F32) + b_ref[...])
    c = jnp.where(ea[:, 0:1] > 0.5, 1.0, 0.0)
    cov_ref[...] = c
    non_ref[...] = 1.0 - c


def _edge_msg_body(hd_ref, hs_ref, e_ref, m_ref, wfd_ref, wfs_ref, wfe_ref,
                   fb_ref, wsd_ref, wss_ref, wse_ref, sb_ref,
                   mlo_ref, mhi_ref):
    hd = hd_ref[...].astype(F32)
    hs = hs_ref[...].astype(F32)
    e = e_ref[...]
    f = (jnp.dot(hd, wfd_ref[...], preferred_element_type=F32)
         + jnp.dot(hs, wfs_ref[...], preferred_element_type=F32)
         + jnp.dot(e, wfe_ref[...], preferred_element_type=F32) + fb_ref[...])
    s = (jnp.dot(hd, wsd_ref[...], preferred_element_type=F32)
         + jnp.dot(hs, wss_ref[...], preferred_element_type=F32)
         + jnp.dot(e, wse_ref[...], preferred_element_type=F32) + sb_ref[...])
    sig = 1.0 / (1.0 + jnp.exp(-f))
    sp = jnp.maximum(s, 0.0) + jnp.log(1.0 + jnp.exp(-jnp.abs(s)))
    msg = m_ref[...] * sig * sp
    mlo_ref[...] = msg[:, :32]
    mhi_ref[...] = msg[:, 32:]


def _node_upd_body(alo_ref, ahi_ref, h_ref, lg_ref, lb_ref, lw_ref, lbias_ref,
                   o_ref, ob_ref):
    h = h_ref[...]
    v = jnp.concatenate([alo_ref[...], ahi_ref[...]], axis=1) + h
    m = jnp.mean(v, axis=1, keepdims=True)
    var = jnp.mean((v - m) * (v - m), axis=1, keepdims=True)
    vn = (v - m) / jnp.sqrt(var + 1e-5) * lg_ref[...] + lb_ref[...]
    o = jnp.dot(vn, lw_ref[...], preferred_element_type=F32) + lbias_ref[...] + h
    o = _elu(o)
    o_ref[...] = o
    ob_ref[...] = o.astype(jnp.bfloat16)


def _head_body(h_ref, x_ref, b_ref, w1_ref, b1_ref, w2_ref, b2_ref, tw_ref,
               na_ref, o_ref):
    pid = pl.program_id(0)
    t = _elu(jnp.dot(h_ref[...], w1_ref[...], preferred_element_type=F32)
             + b1_ref[...])
    t = _elu(jnp.dot(t, w2_ref[...], preferred_element_type=F32) + b2_ref[...])
    lig = jnp.where(x_ref[:, 0:1] > 0.5, 1.0, 0.0)
    sc = jnp.dot(t * lig, tw_ref[...], preferred_element_type=F32)
    onehot = (b_ref[...] == lax.broadcasted_iota(jnp.int32, (NB, G), 1)
              ).astype(F32)
    contrib = lax.dot_general(onehot, sc, (((0,), (0,)), ((), ())))

    @pl.when(pid == 0)
    def _():
        o_ref[...] = jnp.zeros_like(o_ref)

    o_ref[...] += contrib

    @pl.when(pid == NG - 1)
    def _():
        o_ref[...] = o_ref[...] / na_ref[...]


def _const2d(shape):
    return pl.BlockSpec(shape, lambda i: (0, 0))


def _embed_x(x, w_t, b):
    return pl.pallas_call(
        _embed_x_body,
        grid=(NG,),
        in_specs=[pl.BlockSpec((NB, DIN_X), lambda i: (i, 0)),
                  _const2d((DIN_X, DH_X)), _const2d((1, DH_X))],
        out_specs=[pl.BlockSpec((NB, DH_X), lambda i: (i, 0)),
                   pl.BlockSpec((NB, DH_X), lambda i: (i, 0))],
        out_shape=[jax.ShapeDtypeStruct((N, DH_X), F32),
                   jax.ShapeDtypeStruct((N, DH_X), jnp.bfloat16)],
    )(x, w_t, b)


def _embed_e(ea, w_t, b):
    return pl.pallas_call(
        _embed_e_body,
        grid=(EG,),
        in_specs=[pl.BlockSpec((EB, DIN_E), lambda i: (i, 0)),
                  _const2d((DIN_E, DH_E)), _const2d((1, DH_E))],
        out_specs=[pl.BlockSpec((EB, DH_E), lambda i: (i, 0)),
                   pl.BlockSpec((EB, 1), lambda i: (i, 0)),
                   pl.BlockSpec((EB, 1), lambda i: (i, 0))],
        out_shape=[jax.ShapeDtypeStruct((E, DH_E), F32),
                   jax.ShapeDtypeStruct((E, 1), F32),
                   jax.ShapeDtypeStruct((E, 1), F32)],
    )(ea, w_t, b)


def _edge_msg(hd, hs, e, m, wfd, wfs, wfe, fb, wsd, wss, wse, sb):
    return pl.pallas_call(
        _edge_msg_body,
        grid=(EG,),
        in_specs=[pl.BlockSpec((EB, DH_X), lambda i: (i, 0)),
                  pl.BlockSpec((EB, DH_X), lambda i: (i, 0)),
                  pl.BlockSpec((EB, DH_E), lambda i: (i, 0)),
                  pl.BlockSpec((EB, 1), lambda i: (i, 0)),
                  _const2d((DH_X, DH_X)), _const2d((DH_X, DH_X)),
                  _const2d((DH_E, DH_X)), _const2d((1, DH_X)),
                  _const2d((DH_X, DH_X)), _const2d((DH_X, DH_X)),
                  _const2d((DH_E, DH_X)), _const2d((1, DH_X))],
        out_specs=[pl.BlockSpec((EB, 32), lambda i: (i, 0)),
                   pl.BlockSpec((EB, 32), lambda i: (i, 0))],
        out_shape=[jax.ShapeDtypeStruct((E, 32), F32),
                   jax.ShapeDtypeStruct((E, 32), F32)],
    )(hd, hs, e, m, wfd, wfs, wfe, fb, wsd, wss, wse, sb)


def _node_upd(alo, ahi, h, lg, lb, lw_t, lbias):
    return pl.pallas_call(
        _node_upd_body,
        grid=(NG,),
        in_specs=[pl.BlockSpec((NB, 32), lambda i: (i, 0)),
                  pl.BlockSpec((NB, 32), lambda i: (i, 0)),
                  pl.BlockSpec((NB, DH_X), lambda i: (i, 0)),
                  _const2d((1, DH_X)), _const2d((1, DH_X)),
                  _const2d((DH_X, DH_X)), _const2d((1, DH_X))],
        out_specs=[pl.BlockSpec((NB, DH_X), lambda i: (i, 0)),
                   pl.BlockSpec((NB, DH_X), lambda i: (i, 0))],
        out_shape=[jax.ShapeDtypeStruct((N, DH_X), F32),
                   jax.ShapeDtypeStruct((N, DH_X), jnp.bfloat16)],
    )(alo, ahi, h, lg, lb, lw_t, lbias)


def _head(h, x, batch2d, w1_t, b1, w2_t, b2, tw_t, na):
    return pl.pallas_call(
        _head_body,
        grid=(NG,),
        in_specs=[pl.BlockSpec((NB, DH_X), lambda i: (i, 0)),
                  pl.BlockSpec((NB, DIN_X), lambda i: (i, 0)),
                  pl.BlockSpec((NB, 1), lambda i: (i, 0)),
                  _const2d((DH_X, 32)), _const2d((1, 32)),
                  _const2d((32, 16)), _const2d((1, 16)),
                  _const2d((16, 1)), _const2d((G, 1))],
        out_specs=pl.BlockSpec((G, 1), lambda i: (0, 0)),
        out_shape=jax.ShapeDtypeStruct((G, 1), F32),
    )(h, x, batch2d, w1_t, b1, w2_t, b2, tw_t, na)


_MESH = dict(core_axis_name="c", subcore_axis_name="s",
             num_cores=2, num_subcores=16)


def _sc_gather(h, dst2d, src2d):
    BF16 = jnp.bfloat16

    @functools.partial(
        pl.kernel,
        out_type=[jax.ShapeDtypeStruct((E, DH_X), BF16),
                  jax.ShapeDtypeStruct((E, DH_X), BF16)],
        mesh=plsc.VectorSubcoreMesh(**_MESH),
        compiler_params=pltpu.CompilerParams(use_tc_tiling_on_sc=False,
                                             skip_device_barrier=True),
        scratch_types=[pltpu.VMEM((128,), jnp.int32),
                       pltpu.VMEM((128,), jnp.int32),
                       pltpu.VMEM((128,), jnp.int32),
                       pltpu.VMEM((128,), jnp.int32),
                       pltpu.VMEM((128, DH_X), BF16),
                       pltpu.VMEM((128, DH_X), BF16),
                       pltpu.VMEM((128, DH_X), BF16),
                       pltpu.VMEM((128, DH_X), BF16),
                       pltpu.SemaphoreType.DMA,
                       pltpu.SemaphoreType.DMA,
                       pltpu.SemaphoreType.DMA],
    )
    def gk(h_hbm, d2_hbm, s2_hbm, hd_out, hs_out, idx_d0, idx_s0, idx_d1,
           idx_s1, bufd0, bufs0, bufd1, bufs1, semi, semg, semw):
        w = lax.axis_index("s") * 2 + lax.axis_index("c")
        start = 195 * w + jnp.minimum(w, 10)
        cnt = jnp.where(w < 10, 196, 195)

        def row(j):
            return start + jnp.minimum(j, cnt - 1)

        pltpu.async_copy(d2_hbm.at[row(0)], idx_d0, semi)
        pltpu.async_copy(s2_hbm.at[row(0)], idx_s0, semi)
        pltpu.async_copy(d2_hbm.at[row(1)], idx_d1, semi)
        pltpu.async_copy(s2_hbm.at[row(1)], idx_s1, semi)

        def phase(j, idx_d, idx_s, bufd, bufs):
            r = row(j)
            o = pl.ds(r * 128, 128)
            pltpu.make_async_copy(d2_hbm.at[r], idx_d, semi).wait()
            pltpu.make_async_copy(s2_hbm.at[r], idx_s, semi).wait()

            @pl.when(j >= 2)
            def _():
                pltpu.make_async_copy(bufd, hd_out.at[o], semw).wait()
                pltpu.make_async_copy(bufs, hs_out.at[o], semw).wait()

            gd = pltpu.async_copy(h_hbm.at[idx_d], bufd, semg)
            gs = pltpu.async_copy(h_hbm.at[idx_s], bufs, semg)
            gd.wait()
            gs.wait()

            @pl.when(j + 2 < 196)
            def _():
                pltpu.async_copy(d2_hbm.at[row(j + 2)], idx_d, semi)
                pltpu.async_copy(s2_hbm.at[row(j + 2)], idx_s, semi)

            pltpu.async_copy(bufd, hd_out.at[o], semw)
            pltpu.async_copy(bufs, hs_out.at[o], semw)

        def body(jj, carry):
            phase(2 * jj, idx_d0, idx_s0, bufd0, bufs0)
            phase(2 * jj + 1, idx_d1, idx_s1, bufd1, bufs1)
            return carry

        lax.fori_loop(0, 98, body, 0)
        o0 = pl.ds(start * 128, 128)
        pltpu.make_async_copy(bufd0, hd_out.at[o0], semw).wait()
        pltpu.make_async_copy(bufs0, hs_out.at[o0], semw).wait()
        pltpu.make_async_copy(bufd1, hd_out.at[o0], semw).wait()
        pltpu.make_async_copy(bufs1, hs_out.at[o0], semw).wait()

    return gk(h, dst2d, src2d)


def _sc_scatter(mlo, mhi, dst2d, ztile):
    @functools.partial(
        pl.kernel,
        out_type=[jax.ShapeDtypeStruct((N, 32), F32),
                  jax.ShapeDtypeStruct((N, 32), F32)],
        mesh=plsc.VectorSubcoreMesh(**_MESH),
        compiler_params=pltpu.CompilerParams(use_tc_tiling_on_sc=False,
                                             skip_device_barrier=True),
        scratch_types=[pltpu.VMEM((256, 32), F32),
                       pltpu.VMEM((256, 32), F32),
                       pltpu.VMEM((2, 128), jnp.int32),
                       pltpu.VMEM((2, 128), jnp.int32),
                       pltpu.VMEM_SHARED((N, 32), F32),
                       pltpu.SemaphoreType.DMA],
    )
    def sk(mlo_hbm, mhi_hbm, d2_hbm, z_hbm, alo_out, ahi_out,
           mbuf0, mbuf1, midx0, midx1, shared, seml):
        cc = lax.axis_index("c")
        t = lax.axis_index("s")
        pltpu.sync_copy(z_hbm, shared.at[pl.ds(t * NPT, NPT)])
        plsc.subcore_barrier()
        start = 390 * t + jnp.minimum(t, 10)
        cnt = jnp.where(t < 10, 391, 390)
        nblk = 196

        def lbase(b):
            return jnp.minimum(start + 2 * b, ER - 2)

        def issue(b, midx, mbuf):
            lb = lbase(b)
            pltpu.async_copy(d2_hbm.at[pl.ds(lb, 2)], midx, seml)

            @pl.when(cc == 0)
            def _():
                pltpu.async_copy(mlo_hbm.at[pl.ds(lb * 128, 256)], mbuf,
                                 seml)

            @pl.when(cc == 1)
            def _():
                pltpu.async_copy(mhi_hbm.at[pl.ds(lb * 128, 256)], mbuf,
                                 seml)

        issue(0, midx0, mbuf0)
        issue(1, midx1, mbuf1)

        def phase(b, midx, mbuf):
            lb = lbase(b)
            pltpu.make_async_copy(d2_hbm.at[pl.ds(lb, 2)], midx, seml).wait()
            pltpu.make_async_copy(mlo_hbm.at[pl.ds(lb * 128, 256)], mbuf,
                                  seml).wait()
            for k in range(2):
                r = lb + k

                @pl.when((r >= start + 2 * b) & (r < start + cnt))
                def _():
                    pltpu.sync_copy(mbuf.at[pl.ds(k * 128, 128)],
                                    shared.at[midx.at[k]], add=True)

            @pl.when(b + 2 < nblk)
            def _():
                issue(b + 2, midx, mbuf)

        def body(bb, carry):
            phase(2 * bb, midx0, mbuf0)
            phase(2 * bb + 1, midx1, mbuf1)
            return carry

        lax.fori_loop(0, nblk // 2, body, 0)
        plsc.subcore_barrier()

        @pl.when(cc == 0)
        def _():
            pltpu.sync_copy(shared.at[pl.ds(t * NPT, NPT)],
                            alo_out.at[pl.ds(t * NPT, NPT)])

        @pl.when(cc == 1)
        def _():
            pltpu.sync_copy(shared.at[pl.ds(t * NPT, NPT)],
                            ahi_out.at[pl.ds(t * NPT, NPT)])

    return sk(mlo, mhi, dst2d, ztile)


def kernel(x, edge_index, edge_attr, batch, n_atom, embed_x_w, embed_x_b,
           embed_e_w, embed_e_b, cov_lin_f_w, cov_lin_f_b, cov_lin_s_w,
           cov_lin_s_b, cov_ln_g, cov_ln_b, cov_lin_w, cov_lin_b, non_lin_f_w,
           non_lin_f_b, non_lin_s_w, non_lin_s_b, non_ln_g, non_ln_b,
           non_lin_w, non_lin_b, mlp_w1, mlp_b1, mlp_w2, mlp_b2, trans_w):
    dst2d = edge_index[1].reshape(ER, 128)
    src2d = edge_index[0].reshape(ER, 128)
    ztile = jnp.zeros((NPT, 32), F32)

    h, hb = _embed_x(x, embed_x_w.T, embed_x_b.reshape(1, -1))
    e, cov, non = _embed_e(edge_attr, embed_e_w.T, embed_e_b.reshape(1, -1))

    for (m, fw, fb, sw, sb, lg, lb, lw, lbias) in (
        (cov, cov_lin_f_w, cov_lin_f_b, cov_lin_s_w, cov_lin_s_b, cov_ln_g,
         cov_ln_b, cov_lin_w, cov_lin_b),
        (non, non_lin_f_w, non_lin_f_b, non_lin_s_w, non_lin_s_b, non_ln_g,
         non_ln_b, non_lin_w, non_lin_b),
    ):
        hd, hs = _sc_gather(hb, dst2d, src2d)
        mlo, mhi = _edge_msg(
            hd, hs, e, m,
            fw[:, :DH_X].T, fw[:, DH_X:2 * DH_X].T, fw[:, 2 * DH_X:].T,
            fb.reshape(1, -1),
            sw[:, :DH_X].T, sw[:, DH_X:2 * DH_X].T, sw[:, 2 * DH_X:].T,
            sb.reshape(1, -1))
        alo, ahi = _sc_scatter(mlo, mhi, dst2d, ztile)
        h, hb = _node_upd(alo, ahi, h, lg.reshape(1, -1), lb.reshape(1, -1),
                          lw.T, lbias.reshape(1, -1))

    energy = _head(h, x, batch.reshape(N, 1), mlp_w1.T, mlp_b1.reshape(1, -1),
                   mlp_w2.T, mlp_b2.reshape(1, -1), trans_w.T,
                   n_atom.reshape(G, 1))
    return energy.reshape(-1)


# speedup vs baseline: 1.0021x; 1.0015x over previous
import functools

import jax
import jax.numpy as jnp
from jax import lax
from jax.experimental import pallas as pl
from jax.experimental.pallas import tpu as pltpu
from jax.experimental.pallas import tpu_sc as plsc

N = 50000
E = 800000
G = 128
DIN_X = 29
DH_X = 64
DIN_E = 17
DH_E = 32
ER = E // 128
NB = 10000
NG = N // NB
EB = 6400
EG = E // EB
NPT = N // 16
F32 = jnp.float32


def _elu(v):
    return jnp.where(v > 0, v, jnp.exp(jnp.minimum(v, 0.0)) - 1.0)


def _embed_x_body(x_ref, w_ref, b_ref, o_ref, ob_ref):
    h = _elu(jnp.dot(x_ref[...], w_ref[...],
                     preferred_element_type=F32) + b_ref[...])
    o_ref[...] = h
    ob_ref[...] = h.astype(jnp.bfloat16)


def _embed_e_body(ea_ref, w_ref, b_ref, e_ref, cov_ref, non_ref):
    ea = ea_ref[...]
    e_ref[...] = _elu(jnp.dot(ea, w_ref[...],
                              preferred_element_type=F32) + b_ref[...])
    c = jnp.where(ea[:, 0:1] > 0.5, 1.0, 0.0)
    cov_ref[...] = c
    non_ref[...] = 1.0 - c


def _edge_msg_body(hd_ref, hs_ref, e_ref, m_ref, wfd_ref, wfs_ref, wfe_ref,
                   fb_ref, wsd_ref, wss_ref, wse_ref, sb_ref,
                   mlo_ref, mhi_ref):
    hd = hd_ref[...].astype(F32)
    hs = hs_ref[...].astype(F32)
    e = e_ref[...]
    f = (jnp.dot(hd, wfd_ref[...], preferred_element_type=F32)
         + jnp.dot(hs, wfs_ref[...], preferred_element_type=F32)
         + jnp.dot(e, wfe_ref[...], preferred_element_type=F32) + fb_ref[...])
    s = (jnp.dot(hd, wsd_ref[...], preferred_element_type=F32)
         + jnp.dot(hs, wss_ref[...], preferred_element_type=F32)
         + jnp.dot(e, wse_ref[...], preferred_element_type=F32) + sb_ref[...])
    sig = 1.0 / (1.0 + jnp.exp(-f))
    sp = jnp.maximum(s, 0.0) + jnp.log(1.0 + jnp.exp(-jnp.abs(s)))
    msg = m_ref[...] * sig * sp
    mlo_ref[...] = msg[:, :32]
    mhi_ref[...] = msg[:, 32:]


def _node_upd_body(alo_ref, ahi_ref, h_ref, lg_ref, lb_ref, lw_ref, lbias_ref,
                   o_ref, ob_ref):
    h = h_ref[...]
    v = jnp.concatenate([alo_ref[...], ahi_ref[...]], axis=1) + h
    m = jnp.mean(v, axis=1, keepdims=True)
    var = jnp.mean((v - m) * (v - m), axis=1, keepdims=True)
    vn = (v - m) / jnp.sqrt(var + 1e-5) * lg_ref[...] + lb_ref[...]
    o = jnp.dot(vn, lw_ref[...], preferred_element_type=F32) + lbias_ref[...] + h
    o = _elu(o)
    o_ref[...] = o
    ob_ref[...] = o.astype(jnp.bfloat16)


def _node_head_body(alo_ref, ahi_ref, h_ref, lg_ref, lb_ref, lw_ref,
                    lbias_ref, x_ref, b_ref, w1_ref, b1_ref, w2_ref, b2_ref,
                    tw_ref, na_ref, o_ref):
    pid = pl.program_id(0)
    h = h_ref[...]
    v = jnp.concatenate([alo_ref[...], ahi_ref[...]], axis=1) + h
    m = jnp.mean(v, axis=1, keepdims=True)
    var = jnp.mean((v - m) * (v - m), axis=1, keepdims=True)
    vn = (v - m) / jnp.sqrt(var + 1e-5) * lg_ref[...] + lb_ref[...]
    h2 = _elu(jnp.dot(vn, lw_ref[...], preferred_element_type=F32)
              + lbias_ref[...] + h)
    t = _elu(jnp.dot(h2, w1_ref[...], preferred_element_type=F32)
             + b1_ref[...])
    t = _elu(jnp.dot(t, w2_ref[...], preferred_element_type=F32) + b2_ref[...])
    lig = jnp.where(x_ref[:, 0:1] > 0.5, 1.0, 0.0)
    sc = jnp.dot(t * lig, tw_ref[...], preferred_element_type=F32)
    onehot = (b_ref[...] == lax.broadcasted_iota(jnp.int32, (NB, G), 1)
              ).astype(F32)
    contrib = lax.dot_general(onehot, sc, (((0,), (0,)), ((), ())))

    @pl.when(pid == 0)
    def _():
        o_ref[...] = jnp.zeros_like(o_ref)

    o_ref[...] += contrib

    @pl.when(pid == NG - 1)
    def _():
        o_ref[...] = o_ref[...] / na_ref[...]


def _const2d(shape):
    return pl.BlockSpec(shape, lambda i: (0, 0))


def _embed_x(x, w_t, b):
    return pl.pallas_call(
        _embed_x_body,
        grid=(NG,),
        in_specs=[pl.BlockSpec((NB, DIN_X), lambda i: (i, 0)),
                  _const2d((DIN_X, DH_X)), _const2d((1, DH_X))],
        out_specs=[pl.BlockSpec((NB, DH_X), lambda i: (i, 0)),
                   pl.BlockSpec((NB, DH_X), lambda i: (i, 0))],
        out_shape=[jax.ShapeDtypeStruct((N, DH_X), F32),
                   jax.ShapeDtypeStruct((N, DH_X), jnp.bfloat16)],
    )(x, w_t, b)


def _embed_e(ea, w_t, b):
    return pl.pallas_call(
        _embed_e_body,
        grid=(EG,),
        in_specs=[pl.BlockSpec((EB, DIN_E), lambda i: (i, 0)),
                  _const2d((DIN_E, DH_E)), _const2d((1, DH_E))],
        out_specs=[pl.BlockSpec((EB, DH_E), lambda i: (i, 0)),
                   pl.BlockSpec((EB, 1), lambda i: (i, 0)),
                   pl.BlockSpec((EB, 1), lambda i: (i, 0))],
        out_shape=[jax.ShapeDtypeStruct((E, DH_E), F32),
                   jax.ShapeDtypeStruct((E, 1), F32),
                   jax.ShapeDtypeStruct((E, 1), F32)],
    )(ea, w_t, b)


def _edge_msg(hd, hs, e, m, wfd, wfs, wfe, fb, wsd, wss, wse, sb):
    return pl.pallas_call(
        _edge_msg_body,
        grid=(EG,),
        in_specs=[pl.BlockSpec((EB, DH_X), lambda i: (i, 0)),
                  pl.BlockSpec((EB, DH_X), lambda i: (i, 0)),
                  pl.BlockSpec((EB, DH_E), lambda i: (i, 0)),
                  pl.BlockSpec((EB, 1), lambda i: (i, 0)),
                  _const2d((DH_X, DH_X)), _const2d((DH_X, DH_X)),
                  _const2d((DH_E, DH_X)), _const2d((1, DH_X)),
                  _const2d((DH_X, DH_X)), _const2d((DH_X, DH_X)),
                  _const2d((DH_E, DH_X)), _const2d((1, DH_X))],
        out_specs=[pl.BlockSpec((EB, 32), lambda i: (i, 0)),
                   pl.BlockSpec((EB, 32), lambda i: (i, 0))],
        out_shape=[jax.ShapeDtypeStruct((E, 32), F32),
                   jax.ShapeDtypeStruct((E, 32), F32)],
    )(hd, hs, e, m, wfd, wfs, wfe, fb, wsd, wss, wse, sb)


def _node_upd(alo, ahi, h, lg, lb, lw_t, lbias):
    return pl.pallas_call(
        _node_upd_body,
        grid=(NG,),
        in_specs=[pl.BlockSpec((NB, 32), lambda i: (i, 0)),
                  pl.BlockSpec((NB, 32), lambda i: (i, 0)),
                  pl.BlockSpec((NB, DH_X), lambda i: (i, 0)),
                  _const2d((1, DH_X)), _const2d((1, DH_X)),
                  _const2d((DH_X, DH_X)), _const2d((1, DH_X))],
        out_specs=[pl.BlockSpec((NB, DH_X), lambda i: (i, 0)),
                   pl.BlockSpec((NB, DH_X), lambda i: (i, 0))],
        out_shape=[jax.ShapeDtypeStruct((N, DH_X), F32),
                   jax.ShapeDtypeStruct((N, DH_X), jnp.bfloat16)],
    )(alo, ahi, h, lg, lb, lw_t, lbias)


def _node_head(alo, ahi, h, lg, lb, lw_t, lbias, x, batch2d, w1_t, b1, w2_t,
               b2, tw_t, na):
    return pl.pallas_call(
        _node_head_body,
        grid=(NG,),
        compiler_params=pltpu.CompilerParams(
            vmem_limit_bytes=100 * 1024 * 1024),
        in_specs=[pl.BlockSpec((NB, 32), lambda i: (i, 0)),
                  pl.BlockSpec((NB, 32), lambda i: (i, 0)),
                  pl.BlockSpec((NB, DH_X), lambda i: (i, 0)),
                  _const2d((1, DH_X)), _const2d((1, DH_X)),
                  _const2d((DH_X, DH_X)), _const2d((1, DH_X)),
                  pl.BlockSpec((NB, DIN_X), lambda i: (i, 0)),
                  pl.BlockSpec((NB, 1), lambda i: (i, 0)),
                  _const2d((DH_X, 32)), _const2d((1, 32)),
                  _const2d((32, 16)), _const2d((1, 16)),
                  _const2d((16, 1)), _const2d((G, 1))],
        out_specs=pl.BlockSpec((G, 1), lambda i: (0, 0)),
        out_shape=jax.ShapeDtypeStruct((G, 1), F32),
    )(alo, ahi, h, lg, lb, lw_t, lbias, x, batch2d, w1_t, b1, w2_t, b2, tw_t,
      na)


_MESH = dict(core_axis_name="c", subcore_axis_name="s",
             num_cores=2, num_subcores=16)


def _sc_gather(h, dst2d, src2d):
    BF16 = jnp.bfloat16

    @functools.partial(
        pl.kernel,
        out_type=[jax.ShapeDtypeStruct((E, DH_X), BF16),
                  jax.ShapeDtypeStruct((E, DH_X), BF16)],
        mesh=plsc.VectorSubcoreMesh(**_MESH),
        compiler_params=pltpu.CompilerParams(use_tc_tiling_on_sc=False,
                                             skip_device_barrier=True),
        scratch_types=[pltpu.VMEM((128,), jnp.int32),
                       pltpu.VMEM((128,), jnp.int32),
                       pltpu.VMEM((128,), jnp.int32),
                       pltpu.VMEM((128,), jnp.int32),
                       pltpu.VMEM((128, DH_X), BF16),
                       pltpu.VMEM((128, DH_X), BF16),
                       pltpu.VMEM((128, DH_X), BF16),
                       pltpu.VMEM((128, DH_X), BF16),
                       pltpu.SemaphoreType.DMA,
                       pltpu.SemaphoreType.DMA,
                       pltpu.SemaphoreType.DMA],
    )
    def gk(h_hbm, d2_hbm, s2_hbm, hd_out, hs_out, idx_d0, idx_s0, idx_d1,
           idx_s1, bufd0, bufs0, bufd1, bufs1, semi, semg, semw):
        w = lax.axis_index("s") * 2 + lax.axis_index("c")
        start = 195 * w + jnp.minimum(w, 10)
        cnt = jnp.where(w < 10, 196, 195)

        def row(j):
            return start + jnp.minimum(j, cnt - 1)

        pltpu.async_copy(d2_hbm.at[row(0)], idx_d0, semi)
        pltpu.async_copy(s2_hbm.at[row(0)], idx_s0, semi)
        pltpu.async_copy(d2_hbm.at[row(1)], idx_d1, semi)
        pltpu.async_copy(s2_hbm.at[row(1)], idx_s1, semi)

        def phase(j, idx_d, idx_s, bufd, bufs):
            r = row(j)
            o = pl.ds(r * 128, 128)
            pltpu.make_async_copy(d2_hbm.at[r], idx_d, semi).wait()
            pltpu.make_async_copy(s2_hbm.at[r], idx_s, semi).wait()

            @pl.when(j >= 2)
            def _():
                pltpu.make_async_copy(bufd, hd_out.at[o], semw).wait()
                pltpu.make_async_copy(bufs, hs_out.at[o], semw).wait()

            gd = pltpu.async_copy(h_hbm.at[idx_d], bufd, semg)
            gs = pltpu.async_copy(h_hbm.at[idx_s], bufs, semg)
            gd.wait()
            gs.wait()

            @pl.when(j + 2 < 196)
            def _():
                pltpu.async_copy(d2_hbm.at[row(j + 2)], idx_d, semi)
                pltpu.async_copy(s2_hbm.at[row(j + 2)], idx_s, semi)

            pltpu.async_copy(bufd, hd_out.at[o], semw)
            pltpu.async_copy(bufs, hs_out.at[o], semw)

        def body(jj, carry):
            phase(2 * jj, idx_d0, idx_s0, bufd0, bufs0)
            phase(2 * jj + 1, idx_d1, idx_s1, bufd1, bufs1)
            return carry

        lax.fori_loop(0, 98, body, 0)
        o0 = pl.ds(start * 128, 128)
        pltpu.make_async_copy(bufd0, hd_out.at[o0], semw).wait()
        pltpu.make_async_copy(bufs0, hs_out.at[o0], semw).wait()
        pltpu.make_async_copy(bufd1, hd_out.at[o0], semw).wait()
        pltpu.make_async_copy(bufs1, hs_out.at[o0], semw).wait()

    return gk(h, dst2d, src2d)


def _sc_scatter(mlo, mhi, dst2d, ztile):
    @functools.partial(
        pl.kernel,
        out_type=[jax.ShapeDtypeStruct((N, 32), F32),
                  jax.ShapeDtypeStruct((N, 32), F32)],
        mesh=plsc.VectorSubcoreMesh(**_MESH),
        compiler_params=pltpu.CompilerParams(use_tc_tiling_on_sc=False,
                                             skip_device_barrier=True),
        scratch_types=[pltpu.VMEM((256, 32), F32),
                       pltpu.VMEM((256, 32), F32),
                       pltpu.VMEM((2, 128), jnp.int32),
                       pltpu.VMEM((2, 128), jnp.int32),
                       pltpu.VMEM_SHARED((N, 32), F32),
                       pltpu.SemaphoreType.DMA],
    )
    def sk(mlo_hbm, mhi_hbm, d2_hbm, z_hbm, alo_out, ahi_out,
           mbuf0, mbuf1, midx0, midx1, shared, seml):
        cc = lax.axis_index("c")
        t = lax.axis_index("s")
        pltpu.sync_copy(z_hbm, shared.at[pl.ds(t * NPT, NPT)])
        plsc.subcore_barrier()
        start = 390 * t + jnp.minimum(t, 10)
        cnt = jnp.where(t < 10, 391, 390)
        nblk = 196

        def lbase(b):
            return jnp.minimum(start + 2 * b, ER - 2)

        def issue(b, midx, mbuf):
            lb = lbase(b)
            pltpu.async_copy(d2_hbm.at[pl.ds(lb, 2)], midx, seml)

            @pl.when(cc == 0)
            def _():
                pltpu.async_copy(mlo_hbm.at[pl.ds(lb * 128, 256)], mbuf,
                                 seml)

            @pl.when(cc == 1)
            def _():
                pltpu.async_copy(mhi_hbm.at[pl.ds(lb * 128, 256)], mbuf,
                                 seml)

        issue(0, midx0, mbuf0)
        issue(1, midx1, mbuf1)

        def phase(b, midx, mbuf):
            lb = lbase(b)
            pltpu.make_async_copy(d2_hbm.at[pl.ds(lb, 2)], midx, seml).wait()
            pltpu.make_async_copy(mlo_hbm.at[pl.ds(lb * 128, 256)], mbuf,
                                  seml).wait()
            for k in range(2):
                r = lb + k

                @pl.when((r >= start + 2 * b) & (r < start + cnt))
                def _():
                    pltpu.sync_copy(mbuf.at[pl.ds(k * 128, 128)],
                                    shared.at[midx.at[k]], add=True)

            @pl.when(b + 2 < nblk)
            def _():
                issue(b + 2, midx, mbuf)

        def body(bb, carry):
            phase(2 * bb, midx0, mbuf0)
            phase(2 * bb + 1, midx1, mbuf1)
            return carry

        lax.fori_loop(0, nblk // 2, body, 0)
        plsc.subcore_barrier()

        @pl.when(cc == 0)
        def _():
            pltpu.sync_copy(shared.at[pl.ds(t * NPT, NPT)],
                            alo_out.at[pl.ds(t * NPT, NPT)])

        @pl.when(cc == 1)
        def _():
            pltpu.sync_copy(shared.at[pl.ds(t * NPT, NPT)],
                            ahi_out.at[pl.ds(t * NPT, NPT)])

    return sk(mlo, mhi, dst2d, ztile)


def kernel(x, edge_index, edge_attr, batch, n_atom, embed_x_w, embed_x_b,
           embed_e_w, embed_e_b, cov_lin_f_w, cov_lin_f_b, cov_lin_s_w,
           cov_lin_s_b, cov_ln_g, cov_ln_b, cov_lin_w, cov_lin_b, non_lin_f_w,
           non_lin_f_b, non_lin_s_w, non_lin_s_b, non_ln_g, non_ln_b,
           non_lin_w, non_lin_b, mlp_w1, mlp_b1, mlp_w2, mlp_b2, trans_w):
    dst2d = edge_index[1].reshape(ER, 128)
    src2d = edge_index[0].reshape(ER, 128)
    ztile = jnp.zeros((NPT, 32), F32)

    h, hb = _embed_x(x, embed_x_w.T, embed_x_b.reshape(1, -1))
    e, cov, non = _embed_e(edge_attr, embed_e_w.T, embed_e_b.reshape(1, -1))

    def msg_for(fw, fb, sw, sb, hd, hs, m):
        return _edge_msg(
            hd, hs, e, m,
            fw[:, :DH_X].T, fw[:, DH_X:2 * DH_X].T, fw[:, 2 * DH_X:].T,
            fb.reshape(1, -1),
            sw[:, :DH_X].T, sw[:, DH_X:2 * DH_X].T, sw[:, 2 * DH_X:].T,
            sb.reshape(1, -1))

    hd, hs = _sc_gather(hb, dst2d, src2d)
    mlo, mhi = msg_for(cov_lin_f_w, cov_lin_f_b, cov_lin_s_w, cov_lin_s_b,
                       hd, hs, cov)
    alo, ahi = _sc_scatter(mlo, mhi, dst2d, ztile)
    h, hb = _node_upd(alo, ahi, h, cov_ln_g.reshape(1, -1),
                      cov_ln_b.reshape(1, -1), cov_lin_w.T,
                      cov_lin_b.reshape(1, -1))

    hd, hs = _sc_gather(hb, dst2d, src2d)
    mlo, mhi = msg_for(non_lin_f_w, non_lin_f_b, non_lin_s_w, non_lin_s_b,
                       hd, hs, non)
    alo, ahi = _sc_scatter(mlo, mhi, dst2d, ztile)
    energy = _node_head(alo, ahi, h, non_ln_g.reshape(1, -1),
                        non_ln_b.reshape(1, -1), non_lin_w.T,
                        non_lin_b.reshape(1, -1), x, batch.reshape(N, 1),
                        mlp_w1.T, mlp_b1.reshape(1, -1), mlp_w2.T,
                        mlp_b2.reshape(1, -1), trans_w.T, n_atom.reshape(G, 1))
    return energy.reshape(-1)
